# baseline (speedup 1.0000x reference)
import functools

import jax
import jax.numpy as jnp
from jax import lax
from jax.experimental import pallas as pl
from jax.experimental.pallas import tpu as pltpu
from jax.experimental.pallas import tpu_sc as plsc

N = 10000
D = 128
E = 320000
NW = 32
NTILE = 16
CHUNK = 128
CPT = 80
NG = 10
E_PAD = NW * CPT * CHUNK
N_PAD = 10112
RPT = N_PAD // NTILE

_mesh = plsc.VectorSubcoreMesh(core_axis_name="c", subcore_axis_name="s")


def _deg_body(idxr_hbm, ones_hbm, zeros_hbm, degp_hbm, ibuf0, ibuf1, ones_v,
              deg_sh, ssem0, ssem1):
    c = lax.axis_index("c")
    s = lax.axis_index("s")
    wid = s * 2 + c
    r0 = s * RPT
    pltpu.sync_copy(zeros_hbm.at[pl.ds(r0, RPT)], deg_sh.at[pl.ds(r0, RPT)])
    pltpu.sync_copy(ones_hbm, ones_v)
    plsc.subcore_barrier()

    ibufs = (ibuf0, ibuf1)
    ssems = (ssem0, ssem1)

    def issue(g, p):
        pltpu.sync_copy(idxr_hbm.at[wid * NG + g], ibufs[p])
        for t in range(8):
            pltpu.async_copy(ones_v, deg_sh.at[ibufs[p].at[0].at[t]],
                             ssems[p], add=True)

    def drain(p):
        for t in range(8):
            pltpu.make_async_copy(ones_v, deg_sh.at[ibufs[p].at[0].at[t]],
                                  ssems[p]).wait()

    def body(m2, carry):
        g0 = 2 * m2

        @pl.when(m2 > 0)
        def _():
            drain(0)

        issue(g0, 0)

        @pl.when(m2 > 0)
        def _():
            drain(1)

        issue(g0 + 1, 1)
        return carry

    lax.fori_loop(0, NG // 2, body, 0)
    drain(0)
    drain(1)
    plsc.subcore_barrier()
    pltpu.sync_copy(deg_sh.at[pl.ds(r0, RPT)],
                    degp_hbm.at[pl.ds(c * N_PAD + r0, RPT)])


_deg_call = functools.partial(
    pl.kernel,
    out_type=jax.ShapeDtypeStruct((2 * N_PAD, 16), jnp.float32),
    mesh=_mesh,
    scratch_types=[
        pltpu.VMEM((1, 8, CHUNK), jnp.int32),
        pltpu.VMEM((1, 8, CHUNK), jnp.int32),
        pltpu.VMEM((CHUNK, 16), jnp.float32),
        pltpu.VMEM_SHARED((N_PAD, 16), jnp.float32),
        pltpu.SemaphoreType.DMA,
        pltpu.SemaphoreType.DMA,
    ],
    compiler_params=pltpu.CompilerParams(use_tc_tiling_on_sc=False),
)(_deg_body)


def _scat_body(t_hbm, idxg_hbm, zeros_hbm, aggp_hbm,
               ibuf0, ibuf1, dbuf0, dbuf1, agg_sh, isem0, isem1, dsem0, dsem1):
    c = lax.axis_index("c")
    s = lax.axis_index("s")
    wid = s * 2 + c
    r0 = s * RPT
    pltpu.sync_copy(zeros_hbm.at[pl.ds(r0, RPT)], agg_sh.at[pl.ds(r0, RPT)])
    plsc.subcore_barrier()

    ibufs = (ibuf0, ibuf1)
    isems = (isem0, isem1)
    dbufs = (dbuf0, dbuf1)
    dsems = (dsem0, dsem1)

    def istart(g, p):
        pltpu.async_copy(idxg_hbm.at[wid * NG + g], ibufs[p], isems[p])

    def iwait(p):
        pltpu.make_async_copy(idxg_hbm.at[0], ibufs[p], isems[p]).wait()

    H = CHUNK // 4

    def gstart(ibuf, t, p):
        idx = ibuf.at[0].at[t]
        for q in range(4):
            pltpu.async_copy(t_hbm.at[idx.at[pl.ds(q * H, H)]],
                             dbufs[p].at[pl.ds(q * H, H)], dsems[p])

    def process_group(ibuf):
        gstart(ibuf, 0, 0)
        for t in range(8):
            if t < 7:
                gstart(ibuf, t + 1, (t + 1) % 2)
            pltpu.make_async_copy(t_hbm.at[ibuf.at[0].at[t]],
                                  dbufs[t % 2], dsems[t % 2]).wait()
            pltpu.sync_copy(dbufs[t % 2], agg_sh.at[ibuf.at[1].at[t]], add=True)

    istart(0, 0)

    def body(m2, carry):
        g0 = 2 * m2
        istart(g0 + 1, 1)
        iwait(0)
        process_group(ibuf0)

        @pl.when(m2 < NG // 2 - 1)
        def _():
            istart(g0 + 2, 0)

        iwait(1)
        process_group(ibuf1)
        return carry

    lax.fori_loop(0, NG // 2, body, 0)

    plsc.subcore_barrier()
    pltpu.sync_copy(agg_sh.at[pl.ds(r0, RPT)],
                    aggp_hbm.at[pl.ds(c * N_PAD + r0, RPT)])


_scat_call = functools.partial(
    pl.kernel,
    out_type=jax.ShapeDtypeStruct((2 * N_PAD, D), jnp.float32),
    mesh=_mesh,
    scratch_types=[
        pltpu.VMEM((2, 8, CHUNK), jnp.int32),
        pltpu.VMEM((2, 8, CHUNK), jnp.int32),
        pltpu.VMEM((CHUNK, D), jnp.float32),
        pltpu.VMEM((CHUNK, D), jnp.float32),
        pltpu.VMEM_SHARED((N_PAD, D), jnp.float32),
        pltpu.SemaphoreType.DMA,
        pltpu.SemaphoreType.DMA,
        pltpu.SemaphoreType.DMA,
        pltpu.SemaphoreType.DMA,
    ],
    compiler_params=pltpu.CompilerParams(use_tc_tiling_on_sc=False),
)(_scat_body)


_BLK = 1000


def _mm_body(x_ref, degp_ref, wm_ref, t_ref):
    x = x_ref[...]
    d = degp_ref[0] + degp_ref[1]
    dis = jnp.where(d > 0, lax.rsqrt(d), 0.0)[:, 0:1]
    t_ref[...] = jnp.dot(x, wm_ref[...], preferred_element_type=jnp.float32) * dis


def _mm_call(feats, degp, wm_t):
    grid = N // _BLK
    return pl.pallas_call(
        _mm_body,
        grid=(grid,),
        in_specs=[
            pl.BlockSpec((_BLK, D), lambda i: (i, 0)),
            pl.BlockSpec((2, _BLK, 16), lambda i: (0, i, 0)),
            pl.BlockSpec((D, D), lambda i: (0, 0)),
        ],
        out_specs=pl.BlockSpec((_BLK, D), lambda i: (i, 0)),
        out_shape=jax.ShapeDtypeStruct((N, D), jnp.float32),
    )(feats, degp, wm_t)


def _base_body(x_ref, ws_ref, bm_ref, bs_ref, base_ref):
    base_ref[...] = (jnp.dot(x_ref[...], ws_ref[...],
                             preferred_element_type=jnp.float32)
                     + bm_ref[...] + bs_ref[...])


def _base_call(feats, ws_t, bmsg, bskip):
    grid = N // _BLK
    return pl.pallas_call(
        _base_body,
        grid=(grid,),
        in_specs=[
            pl.BlockSpec((_BLK, D), lambda i: (i, 0)),
            pl.BlockSpec((D, D), lambda i: (0, 0)),
            pl.BlockSpec((D,), lambda i: (0,)),
            pl.BlockSpec((D,), lambda i: (0,)),
        ],
        out_specs=pl.BlockSpec((_BLK, D), lambda i: (i, 0)),
        out_shape=jax.ShapeDtypeStruct((N, D), jnp.float32),
    )(feats, ws_t, bmsg, bskip)


def _fin_body(aggp_ref, degp_ref, base_ref, o_ref):
    d = degp_ref[0] + degp_ref[1]
    dis = jnp.where(d > 0, lax.rsqrt(d), 0.0)[:, 0:1]
    o_ref[...] = (aggp_ref[0] + aggp_ref[1]) * dis + base_ref[...]


def _fin_call(aggp, degp, base):
    grid = N // _BLK
    return pl.pallas_call(
        _fin_body,
        grid=(grid,),
        in_specs=[
            pl.BlockSpec((2, _BLK, D), lambda i: (0, i, 0)),
            pl.BlockSpec((2, _BLK, 16), lambda i: (0, i, 0)),
            pl.BlockSpec((_BLK, D), lambda i: (i, 0)),
        ],
        out_specs=pl.BlockSpec((_BLK, D), lambda i: (i, 0)),
        out_shape=jax.ShapeDtypeStruct((N, D), jnp.float32),
    )(aggp, degp, base)


def kernel(feats, edge_index, linear_skip_weight, linear_skip_bias,
           linear_msg_weight, linear_msg_bias):
    row = edge_index[0]
    col = edge_index[1]
    pad = E_PAD - E
    rows = jnp.concatenate([row, jnp.zeros((pad,), jnp.int32)])
    cols = jnp.concatenate([col, jnp.full((pad,), N, jnp.int32)])
    idxg = jnp.stack([rows.reshape(NW, NG, 8, CHUNK),
                      cols.reshape(NW, NG, 8, CHUNK)], axis=2)
    idxg = idxg.reshape(NW * NG, 2, 8, CHUNK)

    ones16 = jnp.ones((CHUNK, 16), jnp.float32)
    zeros16 = jnp.zeros((N_PAD, 16), jnp.float32)
    zerosD = jnp.zeros((N_PAD, D), jnp.float32)

    rows_deg = jnp.concatenate([row, jnp.full((pad,), N, jnp.int32)])
    degp_flat = _deg_call(rows_deg.reshape(NW * NG, 1, 8, CHUNK), ones16, zeros16)
    degp = degp_flat.reshape(2, N_PAD, 16)[:, :N]

    t = _mm_call(feats, degp, linear_msg_weight.T)
    base = _base_call(feats, linear_skip_weight.T, linear_msg_bias,
                      linear_skip_bias)

    aggp_flat = _scat_call(t, idxg, zerosD)
    aggp = aggp_flat.reshape(2, N_PAD, D)[:, :N]

    return _fin_call(aggp, degp, base)


# speedup vs baseline: 1.0449x; 1.0449x over previous
import functools

import jax
import jax.numpy as jnp
from jax import lax
from jax.experimental import pallas as pl
from jax.experimental.pallas import tpu as pltpu
from jax.experimental.pallas import tpu_sc as plsc

N = 10000
D = 128
E = 320000
NW = 32
NTILE = 16
CHUNK = 128
CPT = 80
NG = 10
E_PAD = NW * CPT * CHUNK
N_PAD = 10112
RPT = N_PAD // NTILE

_mesh = plsc.VectorSubcoreMesh(core_axis_name="c", subcore_axis_name="s")


def _deg_body(idxr_hbm, ones_hbm, zeros_hbm, degp_hbm, ibuf0, ibuf1, ones_v,
              deg_sh, ssem0, ssem1):
    c = lax.axis_index("c")
    s = lax.axis_index("s")
    wid = s * 2 + c
    r0 = s * RPT
    pltpu.sync_copy(zeros_hbm.at[pl.ds(r0, RPT)], deg_sh.at[pl.ds(r0, RPT)])
    pltpu.sync_copy(ones_hbm, ones_v)
    plsc.subcore_barrier()

    ibufs = (ibuf0, ibuf1)
    ssems = (ssem0, ssem1)

    def issue(g, p):
        pltpu.sync_copy(idxr_hbm.at[wid * NG + g], ibufs[p])
        for t in range(8):
            pltpu.async_copy(ones_v, deg_sh.at[ibufs[p].at[0].at[t]],
                             ssems[p], add=True)

    def drain(p):
        for t in range(8):
            pltpu.make_async_copy(ones_v, deg_sh.at[ibufs[p].at[0].at[t]],
                                  ssems[p]).wait()

    def body(m2, carry):
        g0 = 2 * m2

        @pl.when(m2 > 0)
        def _():
            drain(0)

        issue(g0, 0)

        @pl.when(m2 > 0)
        def _():
            drain(1)

        issue(g0 + 1, 1)
        return carry

    lax.fori_loop(0, NG // 2, body, 0)
    drain(0)
    drain(1)
    plsc.subcore_barrier()
    pltpu.sync_copy(deg_sh.at[pl.ds(r0, RPT)],
                    degp_hbm.at[pl.ds(c * N_PAD + r0, RPT)])


_deg_call = functools.partial(
    pl.kernel,
    out_type=jax.ShapeDtypeStruct((2 * N_PAD, 16), jnp.float32),
    mesh=_mesh,
    scratch_types=[
        pltpu.VMEM((1, 8, CHUNK), jnp.int32),
        pltpu.VMEM((1, 8, CHUNK), jnp.int32),
        pltpu.VMEM((CHUNK, 16), jnp.float32),
        pltpu.VMEM_SHARED((N_PAD, 16), jnp.float32),
        pltpu.SemaphoreType.DMA,
        pltpu.SemaphoreType.DMA,
    ],
    compiler_params=pltpu.CompilerParams(use_tc_tiling_on_sc=False),
)(_deg_body)


def _scat_body(t_hbm, idxg_hbm, zeros_hbm, aggp_hbm,
               ibuf0, ibuf1, dbuf0, dbuf1, agg_sh, isem0, isem1, dsem0, dsem1):
    c = lax.axis_index("c")
    s = lax.axis_index("s")
    wid = s * 2 + c
    r0 = s * RPT
    pltpu.sync_copy(zeros_hbm.at[pl.ds(r0, RPT)], agg_sh.at[pl.ds(r0, RPT)])
    plsc.subcore_barrier()

    ibufs = (ibuf0, ibuf1)
    isems = (isem0, isem1)
    dbufs = (dbuf0, dbuf1)
    dsems = (dsem0, dsem1)

    def istart(g, p):
        pltpu.async_copy(idxg_hbm.at[wid * NG + g], ibufs[p], isems[p])

    def iwait(p):
        pltpu.make_async_copy(idxg_hbm.at[0], ibufs[p], isems[p]).wait()

    H = CHUNK // 2

    def gstart(ibuf, t, p):
        idx = ibuf.at[0].at[t]
        for q in range(2):
            pltpu.async_copy(t_hbm.at[idx.at[pl.ds(q * H, H)]],
                             dbufs[p].at[pl.ds(q * H, H)], dsems[p])

    def process_group(ibuf):
        gstart(ibuf, 0, 0)
        for t in range(8):
            if t < 7:
                gstart(ibuf, t + 1, (t + 1) % 2)
            pltpu.make_async_copy(t_hbm.at[ibuf.at[0].at[t]],
                                  dbufs[t % 2], dsems[t % 2]).wait()
            pltpu.sync_copy(dbufs[t % 2], agg_sh.at[ibuf.at[1].at[t]], add=True)

    istart(0, 0)

    def body(m2, carry):
        g0 = 2 * m2
        istart(g0 + 1, 1)
        iwait(0)
        process_group(ibuf0)

        @pl.when(m2 < NG // 2 - 1)
        def _():
            istart(g0 + 2, 0)

        iwait(1)
        process_group(ibuf1)
        return carry

    lax.fori_loop(0, NG // 2, body, 0)

    plsc.subcore_barrier()
    pltpu.sync_copy(agg_sh.at[pl.ds(r0, RPT)],
                    aggp_hbm.at[pl.ds(c * N_PAD + r0, RPT)])


_scat_call = functools.partial(
    pl.kernel,
    out_type=jax.ShapeDtypeStruct((2 * N_PAD, D), jnp.float32),
    mesh=_mesh,
    scratch_types=[
        pltpu.VMEM((2, 8, CHUNK), jnp.int32),
        pltpu.VMEM((2, 8, CHUNK), jnp.int32),
        pltpu.VMEM((CHUNK, D), jnp.float32),
        pltpu.VMEM((CHUNK, D), jnp.float32),
        pltpu.VMEM_SHARED((N_PAD, D), jnp.float32),
        pltpu.SemaphoreType.DMA,
        pltpu.SemaphoreType.DMA,
        pltpu.SemaphoreType.DMA,
        pltpu.SemaphoreType.DMA,
    ],
    compiler_params=pltpu.CompilerParams(use_tc_tiling_on_sc=False),
)(_scat_body)


_BLK = 1000


def _mm_body(x_ref, degp_ref, wm_ref, ws_ref, bm_ref, bs_ref, t_ref, base_ref):
    x = x_ref[...]
    d = degp_ref[0] + degp_ref[1]
    dis = jnp.where(d > 0, lax.rsqrt(d), 0.0)[:, 0:1]
    t_ref[...] = jnp.dot(x, wm_ref[...], preferred_element_type=jnp.float32) * dis
    base_ref[...] = (jnp.dot(x, ws_ref[...], preferred_element_type=jnp.float32)
                     + bm_ref[...] + bs_ref[...])


def _mm_call(feats, degp, wm_t, ws_t, bmsg, bskip):
    grid = N // _BLK
    return pl.pallas_call(
        _mm_body,
        grid=(grid,),
        in_specs=[
            pl.BlockSpec((_BLK, D), lambda i: (i, 0)),
            pl.BlockSpec((2, _BLK, 16), lambda i: (0, i, 0)),
            pl.BlockSpec((D, D), lambda i: (0, 0)),
            pl.BlockSpec((D, D), lambda i: (0, 0)),
            pl.BlockSpec((D,), lambda i: (0,)),
            pl.BlockSpec((D,), lambda i: (0,)),
        ],
        out_specs=[
            pl.BlockSpec((_BLK, D), lambda i: (i, 0)),
            pl.BlockSpec((_BLK, D), lambda i: (i, 0)),
        ],
        out_shape=[
            jax.ShapeDtypeStruct((N, D), jnp.float32),
            jax.ShapeDtypeStruct((N, D), jnp.float32),
        ],
    )(feats, degp, wm_t, ws_t, bmsg, bskip)


def _fin_body(aggp_ref, degp_ref, base_ref, o_ref):
    d = degp_ref[0] + degp_ref[1]
    dis = jnp.where(d > 0, lax.rsqrt(d), 0.0)[:, 0:1]
    o_ref[...] = (aggp_ref[0] + aggp_ref[1]) * dis + base_ref[...]


def _fin_call(aggp, degp, base):
    grid = N // _BLK
    return pl.pallas_call(
        _fin_body,
        grid=(grid,),
        in_specs=[
            pl.BlockSpec((2, _BLK, D), lambda i: (0, i, 0)),
            pl.BlockSpec((2, _BLK, 16), lambda i: (0, i, 0)),
            pl.BlockSpec((_BLK, D), lambda i: (i, 0)),
        ],
        out_specs=pl.BlockSpec((_BLK, D), lambda i: (i, 0)),
        out_shape=jax.ShapeDtypeStruct((N, D), jnp.float32),
    )(aggp, degp, base)


def kernel(feats, edge_index, linear_skip_weight, linear_skip_bias,
           linear_msg_weight, linear_msg_bias):
    row = edge_index[0]
    col = edge_index[1]
    pad = E_PAD - E
    rows = jnp.concatenate([row, jnp.zeros((pad,), jnp.int32)])
    cols = jnp.concatenate([col, jnp.full((pad,), N, jnp.int32)])
    idxg = jnp.stack([rows.reshape(NW, NG, 8, CHUNK),
                      cols.reshape(NW, NG, 8, CHUNK)], axis=2)
    idxg = idxg.reshape(NW * NG, 2, 8, CHUNK)

    ones16 = jnp.ones((CHUNK, 16), jnp.float32)
    zeros16 = jnp.zeros((N_PAD, 16), jnp.float32)
    zerosD = jnp.zeros((N_PAD, D), jnp.float32)

    rows_deg = jnp.concatenate([row, jnp.full((pad,), N, jnp.int32)])
    degp_flat = _deg_call(rows_deg.reshape(NW * NG, 1, 8, CHUNK), ones16, zeros16)
    degp = degp_flat.reshape(2, N_PAD, 16)[:, :N]

    t, base = _mm_call(feats, degp, linear_msg_weight.T, linear_skip_weight.T,
                       linear_msg_bias, linear_skip_bias)

    aggp_flat = _scat_call(t, idxg, zerosD)
    aggp = aggp_flat.reshape(2, N_PAD, D)[:, :N]

    return _fin_call(aggp, degp, base)


# speedup vs baseline: 1.0705x; 1.0245x over previous
import functools

import jax
import jax.numpy as jnp
from jax import lax
from jax.experimental import pallas as pl
from jax.experimental.pallas import tpu as pltpu
from jax.experimental.pallas import tpu_sc as plsc

N = 10000
D = 128
E = 320000
NW = 32
NTILE = 16
CHUNK = 128
CPT = 80
NG = 10
E_PAD = NW * CPT * CHUNK
N_PAD = 10112
RPT = N_PAD // NTILE

_mesh = plsc.VectorSubcoreMesh(core_axis_name="c", subcore_axis_name="s")


def _deg_body(idxr_hbm, ones_hbm, zeros_hbm, degp_hbm, ibuf0, ibuf1, ones_v,
              deg_sh, ssem0, ssem1):
    c = lax.axis_index("c")
    s = lax.axis_index("s")
    wid = s * 2 + c
    r0 = s * RPT
    pltpu.sync_copy(zeros_hbm.at[pl.ds(r0, RPT)], deg_sh.at[pl.ds(r0, RPT)])
    pltpu.sync_copy(ones_hbm, ones_v)
    plsc.subcore_barrier()

    ibufs = (ibuf0, ibuf1)
    ssems = (ssem0, ssem1)

    def issue(g, p):
        pltpu.sync_copy(idxr_hbm.at[wid * NG + g], ibufs[p])
        for t in range(8):
            pltpu.async_copy(ones_v, deg_sh.at[ibufs[p].at[0].at[t]],
                             ssems[p], add=True)

    def drain(p):
        for t in range(8):
            pltpu.make_async_copy(ones_v, deg_sh.at[ibufs[p].at[0].at[t]],
                                  ssems[p]).wait()

    def body(m2, carry):
        g0 = 2 * m2

        @pl.when(m2 > 0)
        def _():
            drain(0)

        issue(g0, 0)

        @pl.when(m2 > 0)
        def _():
            drain(1)

        issue(g0 + 1, 1)
        return carry

    lax.fori_loop(0, NG // 2, body, 0)
    drain(0)
    drain(1)
    plsc.subcore_barrier()
    pltpu.sync_copy(deg_sh.at[pl.ds(r0, RPT)],
                    degp_hbm.at[pl.ds(c * N_PAD + r0, RPT)])


_deg_call = functools.partial(
    pl.kernel,
    out_type=jax.ShapeDtypeStruct((2 * N_PAD, 16), jnp.float32),
    mesh=_mesh,
    scratch_types=[
        pltpu.VMEM((1, 8, CHUNK), jnp.int32),
        pltpu.VMEM((1, 8, CHUNK), jnp.int32),
        pltpu.VMEM((CHUNK, 16), jnp.float32),
        pltpu.VMEM_SHARED((N_PAD, 16), jnp.float32),
        pltpu.SemaphoreType.DMA,
        pltpu.SemaphoreType.DMA,
    ],
    compiler_params=pltpu.CompilerParams(use_tc_tiling_on_sc=False),
)(_deg_body)


def _scat_body(t_hbm, idxg_hbm, zeros_hbm, aggp_hbm,
               ibuf0, ibuf1, dbuf0, dbuf1, agg_sh, isem0, isem1, dsem0, dsem1):
    c = lax.axis_index("c")
    s = lax.axis_index("s")
    wid = s * 2 + c
    r0 = s * RPT
    pltpu.sync_copy(zeros_hbm.at[pl.ds(r0, RPT)], agg_sh.at[pl.ds(r0, RPT)])
    plsc.subcore_barrier()

    ibufs = (ibuf0, ibuf1)
    isems = (isem0, isem1)
    dbufs = (dbuf0, dbuf1)
    dsems = (dsem0, dsem1)

    def istart(g, p):
        pltpu.async_copy(idxg_hbm.at[wid * NG + g], ibufs[p], isems[p])

    def iwait(p):
        pltpu.make_async_copy(idxg_hbm.at[0], ibufs[p], isems[p]).wait()

    H = CHUNK // 2

    def gstart(ibuf, t, p):
        idx = ibuf.at[0].at[t]
        for q in range(2):
            pltpu.async_copy(t_hbm.at[idx.at[pl.ds(q * H, H)]],
                             dbufs[p].at[pl.ds(q * H, H)], dsems[p])

    def chunk_fin(ibuf, t):
        pltpu.make_async_copy(t_hbm.at[ibuf.at[0].at[t]],
                              dbufs[t % 2], dsems[t % 2]).wait()
        pltpu.sync_copy(dbufs[t % 2], agg_sh.at[ibuf.at[1].at[t]], add=True)

    istart(0, 0)
    iwait(0)
    gstart(ibuf0, 0, 0)

    def body(m2, carry):
        istart(2 * m2 + 1, 1)
        for t in range(8):
            if t < 7:
                gstart(ibuf0, t + 1, (t + 1) % 2)
            else:
                iwait(1)
                gstart(ibuf1, 0, 0)
            chunk_fin(ibuf0, t)

        @pl.when(m2 < NG // 2 - 1)
        def _():
            istart(2 * m2 + 2, 0)

        for t in range(8):
            if t < 7:
                gstart(ibuf1, t + 1, (t + 1) % 2)
            else:
                @pl.when(m2 < NG // 2 - 1)
                def _():
                    iwait(0)
                    gstart(ibuf0, 0, 0)
            chunk_fin(ibuf1, t)
        return carry

    lax.fori_loop(0, NG // 2, body, 0)

    plsc.subcore_barrier()
    pltpu.sync_copy(agg_sh.at[pl.ds(r0, RPT)],
                    aggp_hbm.at[pl.ds(c * N_PAD + r0, RPT)])


_scat_call = functools.partial(
    pl.kernel,
    out_type=jax.ShapeDtypeStruct((2 * N_PAD, D), jnp.float32),
    mesh=_mesh,
    scratch_types=[
        pltpu.VMEM((2, 8, CHUNK), jnp.int32),
        pltpu.VMEM((2, 8, CHUNK), jnp.int32),
        pltpu.VMEM((CHUNK, D), jnp.float32),
        pltpu.VMEM((CHUNK, D), jnp.float32),
        pltpu.VMEM_SHARED((N_PAD, D), jnp.float32),
        pltpu.SemaphoreType.DMA,
        pltpu.SemaphoreType.DMA,
        pltpu.SemaphoreType.DMA,
        pltpu.SemaphoreType.DMA,
    ],
    compiler_params=pltpu.CompilerParams(use_tc_tiling_on_sc=False),
)(_scat_body)


_BLK = 1000


def _mm_body(x_ref, degp_ref, wm_ref, ws_ref, bm_ref, bs_ref, t_ref, base_ref):
    x = x_ref[...]
    d = degp_ref[0] + degp_ref[1]
    dis = jnp.where(d > 0, lax.rsqrt(d), 0.0)[:, 0:1]
    t_ref[...] = jnp.dot(x, wm_ref[...], preferred_element_type=jnp.float32) * dis
    base_ref[...] = (jnp.dot(x, ws_ref[...], preferred_element_type=jnp.float32)
                     + bm_ref[...] + bs_ref[...])


def _mm_call(feats, degp, wm_t, ws_t, bmsg, bskip):
    grid = N // _BLK
    return pl.pallas_call(
        _mm_body,
        grid=(grid,),
        in_specs=[
            pl.BlockSpec((_BLK, D), lambda i: (i, 0)),
            pl.BlockSpec((2, _BLK, 16), lambda i: (0, i, 0)),
            pl.BlockSpec((D, D), lambda i: (0, 0)),
            pl.BlockSpec((D, D), lambda i: (0, 0)),
            pl.BlockSpec((D,), lambda i: (0,)),
            pl.BlockSpec((D,), lambda i: (0,)),
        ],
        out_specs=[
            pl.BlockSpec((_BLK, D), lambda i: (i, 0)),
            pl.BlockSpec((_BLK, D), lambda i: (i, 0)),
        ],
        out_shape=[
            jax.ShapeDtypeStruct((N, D), jnp.float32),
            jax.ShapeDtypeStruct((N, D), jnp.float32),
        ],
    )(feats, degp, wm_t, ws_t, bmsg, bskip)


def _fin_body(aggp_ref, degp_ref, base_ref, o_ref):
    d = degp_ref[0] + degp_ref[1]
    dis = jnp.where(d > 0, lax.rsqrt(d), 0.0)[:, 0:1]
    o_ref[...] = (aggp_ref[0] + aggp_ref[1]) * dis + base_ref[...]


def _fin_call(aggp, degp, base):
    grid = N // _BLK
    return pl.pallas_call(
        _fin_body,
        grid=(grid,),
        in_specs=[
            pl.BlockSpec((2, _BLK, D), lambda i: (0, i, 0)),
            pl.BlockSpec((2, _BLK, 16), lambda i: (0, i, 0)),
            pl.BlockSpec((_BLK, D), lambda i: (i, 0)),
        ],
        out_specs=pl.BlockSpec((_BLK, D), lambda i: (i, 0)),
        out_shape=jax.ShapeDtypeStruct((N, D), jnp.float32),
    )(aggp, degp, base)


def kernel(feats, edge_index, linear_skip_weight, linear_skip_bias,
           linear_msg_weight, linear_msg_bias):
    row = edge_index[0]
    col = edge_index[1]
    pad = E_PAD - E
    rows = jnp.concatenate([row, jnp.zeros((pad,), jnp.int32)])
    cols = jnp.concatenate([col, jnp.full((pad,), N, jnp.int32)])
    idxg = jnp.stack([rows.reshape(NW, NG, 8, CHUNK),
                      cols.reshape(NW, NG, 8, CHUNK)], axis=2)
    idxg = idxg.reshape(NW * NG, 2, 8, CHUNK)

    ones16 = jnp.ones((CHUNK, 16), jnp.float32)
    zeros16 = jnp.zeros((N_PAD, 16), jnp.float32)
    zerosD = jnp.zeros((N_PAD, D), jnp.float32)

    rows_deg = jnp.concatenate([row, jnp.full((pad,), N, jnp.int32)])
    degp_flat = _deg_call(rows_deg.reshape(NW * NG, 1, 8, CHUNK), ones16, zeros16)
    degp = degp_flat.reshape(2, N_PAD, 16)[:, :N]

    t, base = _mm_call(feats, degp, linear_msg_weight.T, linear_skip_weight.T,
                       linear_msg_bias, linear_skip_bias)

    aggp_flat = _scat_call(t, idxg, zerosD)
    aggp = aggp_flat.reshape(2, N_PAD, D)[:, :N]

    return _fin_call(aggp, degp, base)
